# baseline (speedup 1.0000x reference)
import functools

import jax
import jax.numpy as jnp
from jax import lax
from jax.experimental import pallas as pl
from jax.experimental.pallas import tpu as pltpu
from jax.experimental.pallas import tpu_sc as plsc

N = 10000
E = 320000
D = 128
H = 128
EPS = 1e-08

CW = 144
NC = 2
NS = 16
NW = NC * NS
GB = 128
E_PAD = 327680
NIB = E_PAD // GB
BPW = NIB // NW
NP = 10240
RPT = NP // NS


def _silu(x):
    return x * jax.nn.sigmoid(x)


def _gather_body(hxp, ridx2, cidx2, src_out, tgt_out,
                 ridx_v, cidx_v, srows_v, trows_v, sem1, sem2):
    c = lax.axis_index("c")
    s = lax.axis_index("s")
    wid = s * NC + c
    base = wid * BPW

    def body(i, _):
        blk = base + i
        off = blk * GB
        pltpu.sync_copy(ridx2.at[blk], ridx_v)
        pltpu.sync_copy(cidx2.at[blk], cidx_v)
        cp1 = pltpu.async_copy(hxp.at[ridx_v], srows_v, sem1)
        cp2 = pltpu.async_copy(hxp.at[cidx_v], trows_v, sem2)
        cp1.wait()
        cp2.wait()
        pltpu.sync_copy(srows_v, src_out.at[pl.ds(off, GB)])
        pltpu.sync_copy(trows_v, tgt_out.at[pl.ds(off, GB)])
        return 0

    lax.fori_loop(0, BPW, body, 0)


def _sc_gather(hxp, ridx2, cidx2):
    mesh = plsc.VectorSubcoreMesh(core_axis_name="c", subcore_axis_name="s")
    k = pl.kernel(
        _gather_body,
        out_type=(jax.ShapeDtypeStruct((E_PAD, D), jnp.float32),
                  jax.ShapeDtypeStruct((E_PAD, D), jnp.float32)),
        mesh=mesh,
        scratch_types=[
            pltpu.VMEM((GB,), jnp.int32),
            pltpu.VMEM((GB,), jnp.int32),
            pltpu.VMEM((GB, D), jnp.float32),
            pltpu.VMEM((GB, D), jnp.float32),
            pltpu.SemaphoreType.DMA,
            pltpu.SemaphoreType.DMA,
        ],
    )
    return k(hxp, ridx2, cidx2)


def _scatter_body(packed, sidx2, zeros_hbm, out, idx_v, vals_v, acc):
    c = lax.axis_index("c")
    s = lax.axis_index("s")
    wid = s * NC + c
    base = wid * BPW

    pltpu.sync_copy(zeros_hbm, acc.at[pl.ds(s * RPT, RPT)])
    plsc.subcore_barrier()

    def body(i, _):
        blk = base + i
        off = blk * GB
        pltpu.sync_copy(sidx2.at[blk], idx_v)
        pltpu.sync_copy(packed.at[pl.ds(off, GB)], vals_v)
        pltpu.sync_copy(vals_v, acc.at[idx_v], add=True)
        return 0

    lax.fori_loop(0, BPW, body, 0)
    plsc.subcore_barrier()
    pltpu.sync_copy(acc.at[pl.ds(s * RPT, RPT)], out.at[c, pl.ds(s * RPT, RPT)])


def _sc_scatter(packed, sidx2, zeros_hbm):
    mesh = plsc.VectorSubcoreMesh(core_axis_name="c", subcore_axis_name="s")
    k = pl.kernel(
        _scatter_body,
        out_type=jax.ShapeDtypeStruct((NC, NP, CW), jnp.float32),
        mesh=mesh,
        scratch_types=[
            pltpu.VMEM((GB,), jnp.int32),
            pltpu.VMEM((GB, CW), jnp.float32),
            pltpu.VMEM_SHARED((NP, CW), jnp.float32),
        ],
        compiler_params=pltpu.CompilerParams(use_tc_tiling_on_sc=False),
    )
    return k(packed, sidx2, zeros_hbm)


BE = 2048


def _edge_block(src_ref, tgt_ref, we1s, we1t, we1r, be1, we2, be2,
                wc1, bc1, wc2, out_ref):
    bf = jnp.bfloat16

    def unpack(words):
        xi = lax.bitcast_convert_type(words, jnp.int32)
        lo = lax.bitcast_convert_type(jnp.left_shift(xi, 16), jnp.float32)
        hi = lax.bitcast_convert_type(
            jnp.bitwise_and(xi, jnp.int32(-65536)), jnp.float32)
        return jnp.concatenate([lo, hi], axis=1).astype(bf)

    s = unpack(src_ref[:, :D // 2])
    t = unpack(tgt_ref[:, :D // 2])
    cr = src_ref[:, D // 2:D // 2 + 16]
    cc = tgt_ref[:, D // 2:D // 2 + 16]
    cd = cr - cc
    radial = jnp.sum(cd * cd, axis=1, keepdims=True)
    norm = jnp.sqrt(radial) + EPS
    cdn = cd / norm
    pre1 = (jnp.dot(s, we1s[...].astype(bf), preferred_element_type=jnp.float32)
            + jnp.dot(t, we1t[...].astype(bf), preferred_element_type=jnp.float32)
            + radial * we1r[...] + be1[...])
    e1 = _silu(pre1)
    ef = _silu(jnp.dot(e1.astype(bf), we2[...].astype(bf),
                       preferred_element_type=jnp.float32) + be2[...])
    c1 = _silu(jnp.dot(ef.astype(bf), wc1[...].astype(bf),
                       preferred_element_type=jnp.float32) + bc1[...])
    cwt = jnp.sum(c1 * wc2[...], axis=1, keepdims=True)
    out_ref[:, :D] = ef
    out_ref[:, D:CW] = cdn * cwt


def _tc_edge(src_ext, tgt_ext, we1s, we1t, we1r, be1, we2, be2, wc1, bc1, wc2):
    nblk = E_PAD // BE
    full = lambda shape: pl.BlockSpec(shape, lambda i: (0,) * len(shape))
    return pl.pallas_call(
        _edge_block,
        grid=(nblk,),
        in_specs=[
            pl.BlockSpec((BE, D), lambda i: (i, 0)),
            pl.BlockSpec((BE, D), lambda i: (i, 0)),
            full((D, H)), full((D, H)), full((1, H)), full((1, H)),
            full((H, H)), full((1, H)),
            full((H, H)), full((1, H)), full((1, H)),
        ],
        out_specs=pl.BlockSpec((BE, CW), lambda i: (i, 0)),
        out_shape=jax.ShapeDtypeStruct((E_PAD, CW), jnp.float32),
    )(src_ext, tgt_ext, we1s, we1t, we1r, be1, we2, be2, wc1, bc1, wc2)


BN = 2000


def _node_block(h_ref, cp_ref, agg_ref, wn1h, wn1a, bn1, wn2, bn2,
                hout_ref, cout_ref):
    bf = jnp.bfloat16
    aggf = agg_ref[0] + agg_ref[1]
    agg = aggf[:, :D]
    h = h_ref[...]
    pre = (jnp.dot(h.astype(bf), wn1h[...].astype(bf),
                   preferred_element_type=jnp.float32)
           + jnp.dot(agg.astype(bf), wn1a[...].astype(bf),
                     preferred_element_type=jnp.float32)
           + bn1[...])
    hn = jnp.dot(_silu(pre).astype(bf), wn2[...].astype(bf),
                 preferred_element_type=jnp.float32) + bn2[...]
    hout_ref[...] = h + hn
    cout_ref[...] = cp_ref[...] + aggf[:, D:CW]


def _tc_node(h, coordp, aggp, wn1h, wn1a, bn1, wn2, bn2):
    nblk = N // BN
    full = lambda shape: pl.BlockSpec(shape, lambda i: (0,) * len(shape))
    return pl.pallas_call(
        _node_block,
        grid=(nblk,),
        in_specs=[
            pl.BlockSpec((BN, D), lambda i: (i, 0)),
            pl.BlockSpec((BN, 16), lambda i: (i, 0)),
            pl.BlockSpec((NC, BN, CW), lambda i: (0, i, 0)),
            full((D, H)), full((D, H)), full((1, H)),
            full((H, D)), full((1, D)),
        ],
        out_specs=[
            pl.BlockSpec((BN, D), lambda i: (i, 0)),
            pl.BlockSpec((BN, 16), lambda i: (i, 0)),
        ],
        out_shape=[
            jax.ShapeDtypeStruct((N, D), jnp.float32),
            jax.ShapeDtypeStruct((N, 16), jnp.float32),
        ],
    )(h, coordp, aggp, wn1h, wn1a, bn1, wn2, bn2)


def kernel(h, edge_index, coord, We1, be1, We2, be2, Wn1, bn1, Wn2, bn2,
           Wc1, bc1, Wc2):
    row = edge_index[0]
    col = edge_index[1]

    lo_u = lax.bitcast_convert_type(
        h[:, :D // 2].astype(jnp.bfloat16), jnp.uint16).astype(jnp.uint32)
    hi_u = lax.bitcast_convert_type(
        h[:, D // 2:].astype(jnp.bfloat16), jnp.uint16).astype(jnp.uint32)
    h_pk = lax.bitcast_convert_type((hi_u << 16) | lo_u, jnp.float32)
    hxp = jnp.concatenate(
        [h_pk, coord, jnp.zeros((N, D // 2 - 3), jnp.float32)], axis=1)

    pad = E_PAD - E
    ridx2 = jnp.concatenate([row, jnp.zeros((pad,), jnp.int32)]).reshape(
        NIB, GB)
    cidx2 = jnp.concatenate([col, jnp.zeros((pad,), jnp.int32)]).reshape(
        NIB, GB)
    sidx2 = jnp.concatenate(
        [row, jnp.full((pad,), NP - 1, jnp.int32)]).reshape(NIB, GB)

    src_ext, tgt_ext = _sc_gather(hxp, ridx2, cidx2)

    packed = _tc_edge(
        src_ext, tgt_ext,
        We1[:, :D].T, We1[:, D:2 * D].T, We1[:, 2 * D:].T,
        be1.reshape(1, H), We2.T, be2.reshape(1, H),
        Wc1.T, bc1.reshape(1, H), Wc2.reshape(1, H))

    zeros_hbm = jnp.zeros((RPT, CW), jnp.float32)
    aggp = _sc_scatter(packed, sidx2, zeros_hbm)

    coordp = jnp.pad(coord, ((0, 0), (0, 13)))
    h_out, coutp = _tc_node(
        h, coordp, aggp,
        Wn1[:, :D].T, Wn1[:, D:].T, bn1.reshape(1, H),
        Wn2.T, bn2.reshape(1, D))
    return (h_out, coutp[:, :3])
